# baseline (speedup 1.0000x reference)
import functools

import jax
import jax.numpy as jnp
from jax import lax
from jax.experimental import pallas as pl
from jax.experimental.pallas import tpu as pltpu
from jax.experimental.pallas import tpu_sc as plsc

NC = 2
NS = 16
CHUNK = 80


def _scores_tc(y, w2col, brow, bn=2000):
    n, d = y.shape

    def body(y_ref, w_ref, b_ref, o_ref):
        o_ref[...] = (
            jnp.dot(y_ref[...], w_ref[...], preferred_element_type=jnp.float32)
            + b_ref[...]
        )

    return pl.pallas_call(
        body,
        grid=(n // bn,),
        in_specs=[
            pl.BlockSpec((bn, d), lambda i: (i, 0)),
            pl.BlockSpec((d, 2), lambda i: (0, 0)),
            pl.BlockSpec((1, 2), lambda i: (0, 0)),
        ],
        out_specs=pl.BlockSpec((bn, 2), lambda i: (i, 0)),
        out_shape=jax.ShapeDtypeStruct((n, 2), jnp.float32),
    )(y, w2col, brow)


def _sc_aggregate(s12p, src_flat, dst_flat, y, n, d):
    e = src_flat.shape[0]
    ept = e // (NC * NS)
    nck = ept // CHUNK
    nzc = n // CHUNK
    zc_lo = nzc // NS
    zc_hi = nzc - zc_lo * NS

    mesh = plsc.VectorSubcoreMesh(core_axis_name="c", subcore_axis_name="s")

    @functools.partial(
        pl.kernel,
        mesh=mesh,
        compiler_params=pltpu.CompilerParams(needs_layout_passes=False),
        out_type=[
            jax.ShapeDtypeStruct((NC, n, d), jnp.float32),
            jax.ShapeDtypeStruct((NC * n,), jnp.float32),
        ],
        scratch_types=[
            pltpu.VMEM((n,), jnp.int32),
            pltpu.VMEM((6, CHUNK), jnp.int32),
            pltpu.VMEM((6, CHUNK), jnp.int32),
            pltpu.VMEM((CHUNK, 128), jnp.float32),
            pltpu.VMEM((CHUNK, 128), jnp.float32),
            pltpu.VMEM((CHUNK, 128), jnp.float32),
            pltpu.VMEM((CHUNK,), jnp.float32),
            pltpu.VMEM((CHUNK,), jnp.float32),
            pltpu.VMEM((CHUNK,), jnp.float32),
            pltpu.VMEM_SHARED((n, 128), jnp.float32),
            pltpu.VMEM_SHARED((n,), jnp.float32),
            pltpu.SemaphoreType.DMA,
            [pltpu.SemaphoreType.DMA] * 3,
            [pltpu.SemaphoreType.DMA] * 3,
            [pltpu.SemaphoreType.DMA] * 6,
        ],
    )
    def k(s12_hbm, src_hbm, dst_hbm, y_hbm, agg_out, deg_out,
          s12_v, srcr_v, dstr_v, rows0, rows1, rows2, g0, g1, g2,
          agg_sh, deg_sh, sem_s12, sem_g, sem_sc, sem_i):
        c = lax.axis_index("c")
        s = lax.axis_index("s")
        w = c * NS + s
        ebase = w * ept
        rows = (rows0, rows1, rows2)
        gates = (g0, g1, g2)

        pltpu.async_copy(s12_hbm, s12_v, sem_s12)

        zero16 = jnp.where(lax.iota(jnp.int32, 16) < 0, 1.0, 0.0)

        @plsc.parallel_loop(0, CHUNK, 1, unroll=4)
        def _(j):
            for kk in range(128 // 16):
                rows0[j, pl.ds(kk * 16, 16)] = zero16

        for kk in range(CHUNK // 16):
            g0[pl.ds(kk * 16, 16)] = zero16

        def zrefs(ci):
            return ((rows0, agg_sh.at[pl.ds(ci * CHUNK, CHUNK)]),
                    (g0, deg_sh.at[pl.ds(ci * CHUNK, CHUNK)]))

        my_zc = [s + NS * i for i in range(zc_lo)]

        for ci in my_zc:
            for a, b in zrefs(ci):
                pltpu.async_copy(a, b, sem_sc[0])

        @pl.when(s < zc_hi)
        def _():
            for a, b in zrefs(s + NS * zc_lo):
                pltpu.async_copy(a, b, sem_sc[0])

        for ci in my_zc:
            for a, b in zrefs(ci):
                pltpu.make_async_copy(a, b, sem_sc[0]).wait()

        @pl.when(s < zc_hi)
        def _():
            for a, b in zrefs(s + NS * zc_lo):
                pltpu.make_async_copy(a, b, sem_sc[0]).wait()

        pltpu.make_async_copy(s12_hbm, s12_v, sem_s12).wait()
        plsc.subcore_barrier()

        def idx_src(cc, b6):
            return (src_hbm.at[pl.ds(ebase + cc * CHUNK, CHUNK)],
                    srcr_v.at[b6], sem_i[b6])

        def idx_dst(cc, b6):
            return (dst_hbm.at[pl.ds(ebase + cc * CHUNK, CHUNK)],
                    dstr_v.at[b6], sem_i[b6])

        def start_idx(cc, b6):
            pltpu.async_copy(*idx_src(cc, b6))
            pltpu.async_copy(*idx_dst(cc, b6))

        def wait_idx(cc, b6):
            pltpu.make_async_copy(*idx_src(cc, b6)).wait()
            pltpu.make_async_copy(*idx_dst(cc, b6)).wait()

        def g_refs(b3, b6):
            return (y_hbm.at[srcr_v.at[b6]], rows[b3], sem_g[b3])

        def start_gather(b3, b6):
            pltpu.async_copy(*g_refs(b3, b6))

        def wait_gather(b3, b6):
            pltpu.make_async_copy(*g_refs(b3, b6)).wait()

        def sc_refs(b3, b6):
            return ((rows[b3], agg_sh.at[dstr_v.at[b6]], sem_sc[b3]),
                    (gates[b3], deg_sh.at[dstr_v.at[b6]], sem_sc[b3]))

        def start_scatter(b3, b6):
            for a, bb, ss in sc_refs(b3, b6):
                pltpu.async_copy(a, bb, ss, add=True)

        def drain_scatter(b3, b6):
            for a, bb, ss in sc_refs(b3, b6):
                pltpu.make_async_copy(a, bb, ss).wait()

        himask = jnp.int32(-65536)

        def compute_scale(b3, b6):
            rx = rows[b3]
            gx = gates[b3]
            for i in range(CHUNK // 16):
                si = srcr_v[b6, pl.ds(i * 16, 16)]
                di = dstr_v[b6, pl.ds(i * 16, 16)]
                v1 = plsc.load_gather(s12_v, [si])
                v2 = plsc.load_gather(s12_v, [di])
                a1 = plsc.bitcast(lax.shift_left(v1, 16), jnp.float32)
                a2 = plsc.bitcast(lax.bitwise_and(v2, himask), jnp.float32)
                g = 1.0 / (1.0 + jnp.exp(-(a1 + a2)))
                gx[pl.ds(i * 16, 16)] = g

            @plsc.parallel_loop(0, CHUNK, 1, unroll=8)
            def _(r):
                gg = plsc.load_gather(gx, [lax.broadcast(r, (16,))])
                for kk in range(128 // 16):
                    rx[r, pl.ds(kk * 16, 16)] = rx[r, pl.ds(kk * 16, 16)] * gg

        def run_chunk(cc, k6, drain=True, pre_g=True, pre_i=True):
            b3 = k6 % 3
            bn3 = (b3 + 1) % 3
            wait_gather(b3, k6)
            if drain:
                drain_scatter(bn3, (k6 + 4) % 6)
            if pre_i:
                start_idx(cc + 4, (k6 + 4) % 6)
            if pre_g:
                wait_idx(cc + 1, (k6 + 1) % 6)
                start_gather(bn3, (k6 + 1) % 6)
            compute_scale(b3, k6)
            start_scatter(b3, k6)

        for cc in range(4):
            start_idx(cc, cc)
        wait_idx(0, 0)
        start_gather(0, 0)
        run_chunk(0, 0, drain=False)
        run_chunk(1, 1, drain=False)

        def round6(q, carry):
            cc = 6 * q + 2
            for kk in range(6):
                run_chunk(cc + kk, (2 + kk) % 6)
            return carry

        lax.fori_loop(0, (nck - 11) // 6, round6, 0)

        for cc in range(nck - 9, nck):
            run_chunk(cc, cc % 6, pre_i=(cc + 4 <= nck - 1),
                      pre_g=(cc + 1 <= nck - 1))
        drain_scatter((nck - 2) % 3, (nck - 2) % 6)
        drain_scatter((nck - 1) % 3, (nck - 1) % 6)
        plsc.subcore_barrier()

        def oagg_refs(ci):
            return (agg_sh.at[pl.ds(ci * CHUNK, CHUNK)],
                    agg_out.at[c, pl.ds(ci * CHUNK, CHUNK)])

        for ci in my_zc:
            pltpu.async_copy(*oagg_refs(ci), sem_sc[0])

        @pl.when(s < zc_hi)
        def _():
            pltpu.async_copy(*oagg_refs(s + NS * zc_lo), sem_sc[0])

        def odeg(ci, gbuf, gsem):
            dsl = deg_sh.at[pl.ds(ci * CHUNK, CHUNK)]
            osl = deg_out.at[pl.ds(c * n + ci * CHUNK, CHUNK)]
            pltpu.async_copy(dsl, gbuf, gsem)
            pltpu.make_async_copy(dsl, gbuf, gsem).wait()
            pltpu.async_copy(gbuf, osl, gsem)
            pltpu.make_async_copy(gbuf, osl, gsem).wait()

        for i, ci in enumerate(my_zc):
            odeg(ci, gates[i % 3], sem_sc[1 + i % 2])

        @pl.when(s < zc_hi)
        def _():
            odeg(s + NS * zc_lo, gates[zc_lo % 3], sem_sc[1 + zc_lo % 2])

        for ci in my_zc:
            pltpu.make_async_copy(*oagg_refs(ci), sem_sc[0]).wait()

        @pl.when(s < zc_hi)
        def _():
            pltpu.make_async_copy(*oagg_refs(s + NS * zc_lo), sem_sc[0]).wait()

    return k(s12p, src_flat, dst_flat, y)


def _finish_tc(aggp, degp, w_out, brow, t11, bn=1000):
    _, n, d = aggp.shape

    def body(ap_ref, dp_ref, w_ref, b_ref, t_ref, o_ref):
        a = ap_ref[0] + ap_ref[1]
        ones2 = jnp.ones((NC, 1), jnp.float32)
        deg = jax.lax.dot_general(
            dp_ref[0], ones2, (((0,), (0,)), ((), ())),
            preferred_element_type=jnp.float32)
        h = a / (deg + 1e-6)
        o_ref[...] = jnp.tanh(
            jnp.dot(h, w_ref[...], preferred_element_type=jnp.float32)
            + b_ref[...] + t_ref[0, 0])

    return pl.pallas_call(
        body,
        grid=(n // bn,),
        in_specs=[
            pl.BlockSpec((NC, bn, d), lambda i: (0, i, 0)),
            pl.BlockSpec((1, NC, bn), lambda i: (i, 0, 0)),
            pl.BlockSpec((d, d), lambda i: (0, 0)),
            pl.BlockSpec((1, d), lambda i: (0, 0)),
            pl.BlockSpec(memory_space=pltpu.SMEM),
        ],
        out_specs=pl.BlockSpec((bn, d), lambda i: (i, 0)),
        out_shape=jax.ShapeDtypeStruct((n, d), jnp.float32),
    )(aggp, degp.reshape(NC, n // bn, bn).transpose(1, 0, 2),
      w_out, brow, t11)


def kernel(t, y, edge_index, W_edge, b_edge, W_out, b_out):
    n, d = y.shape
    w2col = jnp.concatenate([W_edge[:d], W_edge[d:]], axis=1)
    brow_e = jnp.concatenate(
        [jnp.zeros((1,), jnp.float32), b_edge]).reshape(1, 2)
    s12 = _scores_tc(y, w2col, brow_e)
    s12p = jax.lax.bitcast_convert_type(s12.astype(jnp.bfloat16), jnp.int32)
    aggp, degf = _sc_aggregate(s12p, edge_index[0], edge_index[1], y, n, d)
    degp = degf.reshape(NC, n)
    return _finish_tc(aggp, degp, W_out, b_out.reshape(1, d),
                      t.reshape(1, 1))


# speedup vs baseline: 1.0139x; 1.0139x over previous
import functools

import jax
import jax.numpy as jnp
from jax import lax
from jax.experimental import pallas as pl
from jax.experimental.pallas import tpu as pltpu
from jax.experimental.pallas import tpu_sc as plsc

NC = 2
NS = 16
CHUNK = 80


def _scores_tc(y, w2col, brow, bn=2000):
    n, d = y.shape

    def body(y_ref, w_ref, b_ref, o_ref):
        o_ref[...] = (
            jnp.dot(y_ref[...], w_ref[...], preferred_element_type=jnp.float32)
            + b_ref[...]
        )

    return pl.pallas_call(
        body,
        grid=(n // bn,),
        in_specs=[
            pl.BlockSpec((bn, d), lambda i: (i, 0)),
            pl.BlockSpec((d, 2), lambda i: (0, 0)),
            pl.BlockSpec((1, 2), lambda i: (0, 0)),
        ],
        out_specs=pl.BlockSpec((bn, 2), lambda i: (i, 0)),
        out_shape=jax.ShapeDtypeStruct((n, 2), jnp.float32),
    )(y, w2col, brow)


def _sc_aggregate(s12p, src_flat, dst_flat, y, n, d):
    e = src_flat.shape[0]
    ept = e // (NC * NS)
    nck = ept // CHUNK
    nzc = n // CHUNK
    zc_lo = nzc // NS
    zc_hi = nzc - zc_lo * NS

    mesh = plsc.VectorSubcoreMesh(core_axis_name="c", subcore_axis_name="s")

    @functools.partial(
        pl.kernel,
        mesh=mesh,
        compiler_params=pltpu.CompilerParams(needs_layout_passes=False),
        out_type=[
            jax.ShapeDtypeStruct((NC, n, d), jnp.float32),
            jax.ShapeDtypeStruct((NC * n,), jnp.float32),
        ],
        scratch_types=[
            pltpu.VMEM((n,), jnp.int32),
            pltpu.VMEM((6, CHUNK), jnp.int32),
            pltpu.VMEM((6, CHUNK), jnp.int32),
            pltpu.VMEM((CHUNK, 128), jnp.float32),
            pltpu.VMEM((CHUNK, 128), jnp.float32),
            pltpu.VMEM((CHUNK, 128), jnp.float32),
            pltpu.VMEM((CHUNK,), jnp.float32),
            pltpu.VMEM((CHUNK,), jnp.float32),
            pltpu.VMEM((CHUNK,), jnp.float32),
            pltpu.VMEM_SHARED((n, 128), jnp.float32),
            pltpu.VMEM_SHARED((n,), jnp.float32),
            pltpu.SemaphoreType.DMA,
            [pltpu.SemaphoreType.DMA] * 3,
            [pltpu.SemaphoreType.DMA] * 3,
            [pltpu.SemaphoreType.DMA] * 6,
        ],
    )
    def k(s12_hbm, src_hbm, dst_hbm, y_hbm, agg_out, deg_out,
          s12_v, srcr_v, dstr_v, rows0, rows1, rows2, g0, g1, g2,
          agg_sh, deg_sh, sem_s12, sem_g, sem_sc, sem_i):
        c = lax.axis_index("c")
        s = lax.axis_index("s")
        w = c * NS + s
        ebase = w * ept
        rows = (rows0, rows1, rows2)
        gates = (g0, g1, g2)

        pltpu.async_copy(s12_hbm, s12_v, sem_s12)

        zero16 = jnp.where(lax.iota(jnp.int32, 16) < 0, 1.0, 0.0)

        @plsc.parallel_loop(0, CHUNK, 1, unroll=4)
        def _(j):
            for kk in range(128 // 16):
                rows0[j, pl.ds(kk * 16, 16)] = zero16

        for kk in range(CHUNK // 16):
            g0[pl.ds(kk * 16, 16)] = zero16

        def zrefs(ci):
            return ((rows0, agg_sh.at[pl.ds(ci * CHUNK, CHUNK)]),
                    (g0, deg_sh.at[pl.ds(ci * CHUNK, CHUNK)]))

        my_zc = [s + NS * i for i in range(zc_lo)]

        for ci in my_zc:
            for a, b in zrefs(ci):
                pltpu.async_copy(a, b, sem_sc[0])

        @pl.when(s < zc_hi)
        def _():
            for a, b in zrefs(s + NS * zc_lo):
                pltpu.async_copy(a, b, sem_sc[0])

        for ci in my_zc:
            for a, b in zrefs(ci):
                pltpu.make_async_copy(a, b, sem_sc[0]).wait()

        @pl.when(s < zc_hi)
        def _():
            for a, b in zrefs(s + NS * zc_lo):
                pltpu.make_async_copy(a, b, sem_sc[0]).wait()

        pltpu.make_async_copy(s12_hbm, s12_v, sem_s12).wait()
        plsc.subcore_barrier()

        def idx_src(cc, b6):
            return (src_hbm.at[pl.ds(ebase + cc * CHUNK, CHUNK)],
                    srcr_v.at[b6], sem_i[b6])

        def idx_dst(cc, b6):
            return (dst_hbm.at[pl.ds(ebase + cc * CHUNK, CHUNK)],
                    dstr_v.at[b6], sem_i[b6])

        def start_idx(cc, b6):
            pltpu.async_copy(*idx_src(cc, b6))
            pltpu.async_copy(*idx_dst(cc, b6))

        def wait_idx(cc, b6):
            pltpu.make_async_copy(*idx_src(cc, b6)).wait()
            pltpu.make_async_copy(*idx_dst(cc, b6)).wait()

        def g_refs(b3, b6):
            return (y_hbm.at[srcr_v.at[b6]], rows[b3], sem_g[b3])

        def start_gather(b3, b6):
            pltpu.async_copy(*g_refs(b3, b6))

        def wait_gather(b3, b6):
            pltpu.make_async_copy(*g_refs(b3, b6)).wait()

        def sc_refs(b3, b6):
            return ((rows[b3], agg_sh.at[dstr_v.at[b6]], sem_sc[b3]),
                    (gates[b3], deg_sh.at[dstr_v.at[b6]], sem_sc[b3]))

        def start_scatter(b3, b6):
            for a, bb, ss in sc_refs(b3, b6):
                pltpu.async_copy(a, bb, ss, add=True)

        def drain_scatter(b3, b6):
            for a, bb, ss in sc_refs(b3, b6):
                pltpu.make_async_copy(a, bb, ss).wait()

        himask = jnp.int32(-65536)

        def compute_scale(b3, b6):
            rx = rows[b3]
            gx = gates[b3]
            for i in range(CHUNK // 16):
                si = srcr_v[b6, pl.ds(i * 16, 16)]
                di = dstr_v[b6, pl.ds(i * 16, 16)]
                v1 = plsc.load_gather(s12_v, [si])
                v2 = plsc.load_gather(s12_v, [di])
                a1 = plsc.bitcast(lax.shift_left(v1, 16), jnp.float32)
                a2 = plsc.bitcast(lax.bitwise_and(v2, himask), jnp.float32)
                g = 1.0 / (1.0 + jnp.exp(-(a1 + a2)))
                gx[pl.ds(i * 16, 16)] = g

            @plsc.parallel_loop(0, CHUNK, 1, unroll=4)
            def _(r):
                gg = plsc.load_gather(gx, [lax.broadcast(r, (16,))])
                for kk in range(128 // 16):
                    rx[r, pl.ds(kk * 16, 16)] = rx[r, pl.ds(kk * 16, 16)] * gg

        def run_chunk(cc, k6, drain=True, pre_g=True, pre_i=True):
            b3 = k6 % 3
            bn3 = (b3 + 1) % 3
            wait_gather(b3, k6)
            if drain:
                drain_scatter(bn3, (k6 + 4) % 6)
            if pre_i:
                start_idx(cc + 4, (k6 + 4) % 6)
            if pre_g:
                wait_idx(cc + 1, (k6 + 1) % 6)
                start_gather(bn3, (k6 + 1) % 6)
            compute_scale(b3, k6)
            start_scatter(b3, k6)

        for cc in range(4):
            start_idx(cc, cc)
        wait_idx(0, 0)
        start_gather(0, 0)
        run_chunk(0, 0, drain=False)
        run_chunk(1, 1, drain=False)

        def round6(q, carry):
            cc = 6 * q + 2
            for kk in range(6):
                run_chunk(cc + kk, (2 + kk) % 6)
            return carry

        lax.fori_loop(0, (nck - 11) // 6, round6, 0)

        for cc in range(nck - 9, nck):
            run_chunk(cc, cc % 6, pre_i=(cc + 4 <= nck - 1),
                      pre_g=(cc + 1 <= nck - 1))
        drain_scatter((nck - 2) % 3, (nck - 2) % 6)
        drain_scatter((nck - 1) % 3, (nck - 1) % 6)
        plsc.subcore_barrier()

        def oagg_refs(ci):
            return (agg_sh.at[pl.ds(ci * CHUNK, CHUNK)],
                    agg_out.at[c, pl.ds(ci * CHUNK, CHUNK)])

        for ci in my_zc:
            pltpu.async_copy(*oagg_refs(ci), sem_sc[0])

        @pl.when(s < zc_hi)
        def _():
            pltpu.async_copy(*oagg_refs(s + NS * zc_lo), sem_sc[0])

        def odeg(ci, gbuf, gsem):
            dsl = deg_sh.at[pl.ds(ci * CHUNK, CHUNK)]
            osl = deg_out.at[pl.ds(c * n + ci * CHUNK, CHUNK)]
            pltpu.async_copy(dsl, gbuf, gsem)
            pltpu.make_async_copy(dsl, gbuf, gsem).wait()
            pltpu.async_copy(gbuf, osl, gsem)
            pltpu.make_async_copy(gbuf, osl, gsem).wait()

        for i, ci in enumerate(my_zc):
            odeg(ci, gates[i % 3], sem_sc[1 + i % 2])

        @pl.when(s < zc_hi)
        def _():
            odeg(s + NS * zc_lo, gates[zc_lo % 3], sem_sc[1 + zc_lo % 2])

        for ci in my_zc:
            pltpu.make_async_copy(*oagg_refs(ci), sem_sc[0]).wait()

        @pl.when(s < zc_hi)
        def _():
            pltpu.make_async_copy(*oagg_refs(s + NS * zc_lo), sem_sc[0]).wait()

    return k(s12p, src_flat, dst_flat, y)


def _finish_tc(aggp, degp, w_out, brow, t11, bn=1000):
    _, n, d = aggp.shape

    def body(ap_ref, dp_ref, w_ref, b_ref, t_ref, o_ref):
        a = ap_ref[0] + ap_ref[1]
        ones2 = jnp.ones((NC, 1), jnp.float32)
        deg = jax.lax.dot_general(
            dp_ref[0], ones2, (((0,), (0,)), ((), ())),
            preferred_element_type=jnp.float32)
        h = a / (deg + 1e-6)
        o_ref[...] = jnp.tanh(
            jnp.dot(h, w_ref[...], preferred_element_type=jnp.float32)
            + b_ref[...] + t_ref[0, 0])

    return pl.pallas_call(
        body,
        grid=(n // bn,),
        in_specs=[
            pl.BlockSpec((NC, bn, d), lambda i: (0, i, 0)),
            pl.BlockSpec((1, NC, bn), lambda i: (i, 0, 0)),
            pl.BlockSpec((d, d), lambda i: (0, 0)),
            pl.BlockSpec((1, d), lambda i: (0, 0)),
            pl.BlockSpec(memory_space=pltpu.SMEM),
        ],
        out_specs=pl.BlockSpec((bn, d), lambda i: (i, 0)),
        out_shape=jax.ShapeDtypeStruct((n, d), jnp.float32),
    )(aggp, degp.reshape(NC, n // bn, bn).transpose(1, 0, 2),
      w_out, brow, t11)


def kernel(t, y, edge_index, W_edge, b_edge, W_out, b_out):
    n, d = y.shape
    w2col = jnp.concatenate([W_edge[:d], W_edge[d:]], axis=1)
    brow_e = jnp.concatenate(
        [jnp.zeros((1,), jnp.float32), b_edge]).reshape(1, 2)
    s12 = _scores_tc(y, w2col, brow_e)
    s12p = jax.lax.bitcast_convert_type(s12.astype(jnp.bfloat16), jnp.int32)
    aggp, degf = _sc_aggregate(s12p, edge_index[0], edge_index[1], y, n, d)
    degp = degf.reshape(NC, n)
    return _finish_tc(aggp, degp, W_out, b_out.reshape(1, d),
                      t.reshape(1, 1))
